# Initial kernel scaffold; baseline (speedup 1.0000x reference)
import jax
import jax.numpy as jnp
from jax.experimental import pallas as pl


def kernel(xyz):
    raise NotImplementedError("write your pallas kernel here")


# speedup vs baseline: 11.0492x; 11.0492x over previous
import functools

import jax
import jax.numpy as jnp
from jax import lax
from jax.experimental import pallas as pl
from jax.experimental.pallas import tpu as pltpu
from jax.experimental.pallas import tpu_sc as plsc

B = 16
N = 8192
G = 512
M = 32
TG = 256
BIG = 3.0e38


def _fps_body(xp_ref, yp_ref, zp_ref, cx_ref, cy_ref, cz_ref, dist_ref):
    lane = lax.broadcasted_iota(jnp.int32, (B, N), 1)
    gcol = lax.broadcasted_iota(jnp.int32, (B, G), 1)
    dist_ref[...] = jnp.full((B, N), 1e10, jnp.float32)

    def step(t, carry):
        far, ax, ay, az = carry
        xp = xp_ref[...]
        yp = yp_ref[...]
        zp = zp_ref[...]
        oh = lane == far
        cx = jnp.sum(jnp.where(oh, xp, 0.0), axis=1, keepdims=True)
        cy = jnp.sum(jnp.where(oh, yp, 0.0), axis=1, keepdims=True)
        cz = jnp.sum(jnp.where(oh, zp, 0.0), axis=1, keepdims=True)
        sel = gcol == t
        ax = jnp.where(sel, cx, ax)
        ay = jnp.where(sel, cy, ay)
        az = jnp.where(sel, cz, az)
        d = (xp - cx) ** 2 + (yp - cy) ** 2 + (zp - cz) ** 2
        dist = jnp.minimum(dist_ref[...], d)
        dist_ref[...] = dist
        m = jnp.max(dist, axis=1, keepdims=True)
        far_new = jnp.min(jnp.where(dist == m, lane, N), axis=1, keepdims=True)
        return far_new, ax, ay, az

    zc = jnp.zeros((B, G), jnp.float32)
    _, ax, ay, az = lax.fori_loop(
        0, G, step, (jnp.zeros((B, 1), jnp.int32), zc, zc, zc))
    cx_ref[...] = ax
    cy_ref[...] = ay
    cz_ref[...] = az


def _fps(xp, yp, zp, *, interpret=False):
    out = jax.ShapeDtypeStruct((B, G), jnp.float32)
    return pl.pallas_call(
        _fps_body,
        out_shape=(out, out, out),
        scratch_shapes=[pltpu.VMEM((B, N), jnp.float32)],
        interpret=interpret,
    )(xp, yp, zp)


def _knn_body(x_ref, c_ref, q2_ref, r2_ref, idx_ref, key_ref):
    e = lax.dot_general(c_ref[0], x_ref[0], (((1,), (1,)), ((), ())))
    key_ref[...] = jnp.sqrt(
        jnp.maximum((q2_ref[...] + r2_ref[0]) - 2.0 * e, 0.0))
    lane = lax.broadcasted_iota(jnp.int32, (1, N), 1)
    mcol = lax.broadcasted_iota(jnp.int32, (TG, M), 1)

    def step(t, acc):
        key = key_ref[...]
        m = jnp.min(key, axis=1, keepdims=True)
        cand = jnp.where(key == m, lane, N)
        idx = jnp.min(cand, axis=1, keepdims=True)
        acc = jnp.where(mcol == t, idx, acc)
        key_ref[...] = jnp.where(lane == idx, BIG, key)
        return acc

    idx_ref[...] = lax.fori_loop(0, M, step, jnp.zeros((TG, M), jnp.int32))


def _knn(xyz, cent3, q2, r2, *, interpret=False):
    gt = G // TG
    col = lambda b, g: (b * gt + g, 0)
    return pl.pallas_call(
        _knn_body,
        grid=(B, gt),
        in_specs=[
            pl.BlockSpec((1, N, 3), lambda b, g: (b, 0, 0)),
            pl.BlockSpec((1, TG, 3), lambda b, g: (b, g, 0)),
            pl.BlockSpec((TG, 1), col),
            pl.BlockSpec((1, 1, N), lambda b, g: (b, 0, 0)),
        ],
        out_specs=pl.BlockSpec((TG, M), col),
        out_shape=jax.ShapeDtypeStruct((B * G, M), jnp.int32),
        scratch_shapes=[pltpu.VMEM((TG, N), jnp.float32)],
        interpret=interpret,
    )(xyz, cent3, q2, r2)


GH = G // 2


def _gather_body(xyzp, cent, idxT, out,
                 xv, yv, zv, cxv, cyv, czv, iv, oxv, oyv, ozv):
    wid = lax.axis_index("s") * 2 + lax.axis_index("c")
    b = wid // 2
    base = (wid % 2) * GH
    pltpu.sync_copy(xyzp.at[0, b], xv)
    pltpu.sync_copy(xyzp.at[1, b], yv)
    pltpu.sync_copy(xyzp.at[2, b], zv)
    pltpu.sync_copy(cent.at[0, b, pl.ds(base, GH)], cxv)
    pltpu.sync_copy(cent.at[1, b, pl.ds(base, GH)], cyv)
    pltpu.sync_copy(cent.at[2, b, pl.ds(base, GH)], czv)
    pltpu.sync_copy(idxT.at[b, :, pl.ds(base, GH)], iv)

    def step(i, carry):
        m = i & (M - 1)
        g0 = (i >> 5) * 16
        ix = iv[m, pl.ds(g0, 16)]
        cxs = cxv[pl.ds(g0, 16)]
        cys = cyv[pl.ds(g0, 16)]
        czs = czv[pl.ds(g0, 16)]
        oxv[m, pl.ds(g0, 16)] = plsc.load_gather(xv, [ix]) - cxs
        oyv[m, pl.ds(g0, 16)] = plsc.load_gather(yv, [ix]) - cys
        ozv[m, pl.ds(g0, 16)] = plsc.load_gather(zv, [ix]) - czs
        return carry

    lax.fori_loop(0, M * GH // 16, step, 0)
    pltpu.sync_copy(oxv, out.at[0, b, :, pl.ds(base, GH)])
    pltpu.sync_copy(oyv, out.at[1, b, :, pl.ds(base, GH)])
    pltpu.sync_copy(ozv, out.at[2, b, :, pl.ds(base, GH)])


def _gather(xyzp, cent, idxT):
    mesh = plsc.VectorSubcoreMesh(core_axis_name="c", subcore_axis_name="s")
    fn = pl.kernel(
        _gather_body,
        out_type=jax.ShapeDtypeStruct((3, B, M, G), jnp.float32),
        mesh=mesh,
        compiler_params=pltpu.CompilerParams(needs_layout_passes=False),
        scratch_types=[
            pltpu.VMEM((N,), jnp.float32),
            pltpu.VMEM((N,), jnp.float32),
            pltpu.VMEM((N,), jnp.float32),
            pltpu.VMEM((GH,), jnp.float32),
            pltpu.VMEM((GH,), jnp.float32),
            pltpu.VMEM((GH,), jnp.float32),
            pltpu.VMEM((M, GH), jnp.int32),
            pltpu.VMEM((M, GH), jnp.float32),
            pltpu.VMEM((M, GH), jnp.float32),
            pltpu.VMEM((M, GH), jnp.float32),
        ],
    )
    return fn(xyzp, cent, idxT)


def kernel(xyz):
    xyzp = jnp.transpose(xyz, (2, 0, 1))
    xp, yp, zp = xyzp[0], xyzp[1], xyzp[2]
    cx, cy, cz = _fps(xp, yp, zp)
    center = jnp.stack([cx, cy, cz], axis=-1)
    q2 = jnp.sum(center ** 2, axis=-1).reshape(B * G, 1)
    r2 = jnp.sum(xyz ** 2, axis=-1).reshape(B, 1, N)
    idx = _knn(xyz, center, q2, r2)
    idxT = jnp.transpose(idx.reshape(B, G, M), (0, 2, 1))
    cent = jnp.stack([cx, cy, cz], axis=0)
    nbh = _gather(xyzp, cent, idxT)
    neighborhood = jnp.transpose(nbh, (1, 3, 2, 0))
    return neighborhood, center
